# baseline (speedup 1.0000x reference)
import jax
import jax.numpy as jnp
from jax.experimental import pallas as pl
from jax.experimental.pallas import tpu as pltpu

BM = 256
BK = 1024


def _body(t_ref, x_ref, o_ref, xt_ref, acc_ref):
    k = pl.program_id(0)
    m = pl.program_id(1)
    nk = pl.num_programs(0)
    B = x_ref.shape[0]
    C = x_ref.shape[2]

    @pl.when(m == 0)
    def _():
        for b in range(B):
            xt_ref[:, b * C:(b + 1) * C] = x_ref[b].astype(jnp.bfloat16)

    t = t_ref[...].astype(jnp.bfloat16)
    partial = jnp.dot(t, xt_ref[...], preferred_element_type=jnp.float32)

    msl = pl.ds(m * BM, BM)

    @pl.when(k == 0)
    def _():
        acc_ref[msl, :] = partial

    @pl.when(jnp.logical_and(k > 0, k < nk - 1))
    def _():
        acc_ref[msl, :] += partial

    @pl.when(k == nk - 1)
    def _():
        final = acc_ref[msl, :] + partial
        for b in range(B):
            o_ref[b, msl, :] = final[:, b * C:(b + 1) * C]


@jax.jit
def kernel(x, transform):
    B, V_in, C = x.shape
    V_out = transform.shape[0]
    N = B * C

    return pl.pallas_call(
        _body,
        grid=(V_in // BK, V_out // BM),
        in_specs=[
            pl.BlockSpec((BM, BK), lambda k, m: (m, k)),
            pl.BlockSpec((B, BK, C), lambda k, m: (0, k, 0)),
        ],
        out_specs=pl.BlockSpec((B, V_out, C), lambda k, m: (0, 0, 0)),
        out_shape=jax.ShapeDtypeStruct((B, V_out, C), jnp.float32),
        scratch_shapes=[
            pltpu.VMEM((BK, N), jnp.bfloat16),
            pltpu.VMEM((V_out, N), jnp.float32),
        ],
        compiler_params=pltpu.CompilerParams(
            dimension_semantics=("arbitrary", "arbitrary"),
        ),
    )(transform, x)


# speedup vs baseline: 1.2333x; 1.2333x over previous
import jax
import jax.numpy as jnp
from jax.experimental import pallas as pl
from jax.experimental.pallas import tpu as pltpu

BM = 256


def _body(t_ref, x_ref, o_ref):
    t = t_ref[...].astype(jnp.bfloat16)
    o_ref[...] = jnp.dot(t, x_ref[...], preferred_element_type=jnp.float32)


@jax.jit
def kernel(x, transform):
    B, V_in, C = x.shape
    V_out = transform.shape[0]
    N = B * C
    xt = x.astype(jnp.bfloat16).transpose(1, 0, 2).reshape(V_in, N)

    out2d = pl.pallas_call(
        _body,
        grid=(V_out // BM,),
        in_specs=[
            pl.BlockSpec((BM, V_in), lambda m: (m, 0)),
            pl.BlockSpec((V_in, N), lambda m: (0, 0)),
        ],
        out_specs=pl.BlockSpec((BM, N), lambda m: (m, 0)),
        out_shape=jax.ShapeDtypeStruct((V_out, N), jnp.float32),
        compiler_params=pltpu.CompilerParams(
            dimension_semantics=("arbitrary",),
        ),
    )(transform, xt)

    return out2d.reshape(V_out, B, C).transpose(1, 0, 2)


# speedup vs baseline: 1.3674x; 1.1088x over previous
import jax
import jax.numpy as jnp
from jax.experimental import pallas as pl
from jax.experimental.pallas import tpu as pltpu

BM = 256


def _body(t_ref, x_ref, o_ref):
    B = o_ref.shape[0]
    C = o_ref.shape[2]
    t = t_ref[...].astype(jnp.bfloat16)
    partial = jnp.dot(t, x_ref[...], preferred_element_type=jnp.float32)
    for b in range(B):
        o_ref[b, :, :] = partial[:, b * C:(b + 1) * C]


@jax.jit
def kernel(x, transform):
    B, V_in, C = x.shape
    V_out = transform.shape[0]
    N = B * C
    xt = x.astype(jnp.bfloat16).transpose(1, 0, 2).reshape(V_in, N)

    return pl.pallas_call(
        _body,
        grid=(V_out // BM,),
        in_specs=[
            pl.BlockSpec((BM, V_in), lambda m: (m, 0)),
            pl.BlockSpec((V_in, N), lambda m: (0, 0)),
        ],
        out_specs=pl.BlockSpec((B, BM, C), lambda m: (0, m, 0)),
        out_shape=jax.ShapeDtypeStruct((B, V_out, C), jnp.float32),
        compiler_params=pltpu.CompilerParams(
            dimension_semantics=("arbitrary",),
        ),
    )(transform, xt)


# speedup vs baseline: 1.5598x; 1.1407x over previous
import jax
import jax.numpy as jnp
from jax.experimental import pallas as pl
from jax.experimental.pallas import tpu as pltpu

BM = 256
CK = 1024


def _body(t_ref, x_ref, o_ref, xt_ref, cbuf_ref, sems):
    m = pl.program_id(0)
    B = o_ref.shape[0]
    C = o_ref.shape[2]
    V_in = xt_ref.shape[0]
    nchunk = V_in // CK

    @pl.when(m == 0)
    def _():
        def chunk_copy(i, slot):
            return pltpu.make_async_copy(
                x_ref.at[:, pl.ds(i * CK, CK), :],
                cbuf_ref.at[slot],
                sems.at[slot],
            )

        chunk_copy(0, 0).start()
        for i in range(nchunk):
            slot = i % 2
            if i + 1 < nchunk:
                chunk_copy(i + 1, (i + 1) % 2).start()
            chunk_copy(i, slot).wait()
            for b in range(B):
                xt_ref[pl.ds(i * CK, CK), b * C:(b + 1) * C] = (
                    cbuf_ref[slot, b].astype(jnp.bfloat16))

    t = t_ref[...].astype(jnp.bfloat16)
    partial = jnp.dot(t, xt_ref[...], preferred_element_type=jnp.float32)
    for b in range(B):
        o_ref[b, :, :] = partial[:, b * C:(b + 1) * C]


@jax.jit
def kernel(x, transform):
    B, V_in, C = x.shape
    V_out = transform.shape[0]
    N = B * C

    return pl.pallas_call(
        _body,
        grid=(V_out // BM,),
        in_specs=[
            pl.BlockSpec((BM, V_in), lambda m: (m, 0)),
            pl.BlockSpec(memory_space=pltpu.MemorySpace.HBM),
        ],
        out_specs=pl.BlockSpec((B, BM, C), lambda m: (0, m, 0)),
        out_shape=jax.ShapeDtypeStruct((B, V_out, C), jnp.float32),
        scratch_shapes=[
            pltpu.VMEM((V_in, N), jnp.bfloat16),
            pltpu.VMEM((2, B, CK, C), jnp.float32),
            pltpu.SemaphoreType.DMA((2,)),
        ],
        compiler_params=pltpu.CompilerParams(
            dimension_semantics=("arbitrary",),
        ),
    )(transform, x)


# speedup vs baseline: 1.7007x; 1.0903x over previous
import jax
import jax.numpy as jnp
from jax.experimental import pallas as pl
from jax.experimental.pallas import tpu as pltpu

BM = 256
CK = 1024


def _body(t_ref, x_ref, o_ref, xt_ref, cbuf_ref, sems):
    m = pl.program_id(0)
    B = o_ref.shape[0]
    C = o_ref.shape[2]
    V_in = xt_ref.shape[0]
    nchunk = V_in // CK

    def write_out(partial):
        for b in range(B):
            o_ref[b, :, :] = partial[:, b * C:(b + 1) * C]

    @pl.when(m == 0)
    def _():
        def chunk_copy(i, slot):
            return pltpu.make_async_copy(
                x_ref.at[:, pl.ds(i * CK, CK), :],
                cbuf_ref.at[slot],
                sems.at[slot],
            )

        chunk_copy(0, 0).start()
        acc = None
        for i in range(nchunk):
            slot = i % 2
            if i + 1 < nchunk:
                chunk_copy(i + 1, (i + 1) % 2).start()
            chunk_copy(i, slot).wait()
            for b in range(B):
                xt_ref[pl.ds(i * CK, CK), b * C:(b + 1) * C] = (
                    cbuf_ref[slot, b].astype(jnp.bfloat16))
            tc = t_ref[:, i * CK:(i + 1) * CK].astype(jnp.bfloat16)
            d = jnp.dot(tc, xt_ref[pl.ds(i * CK, CK), :],
                        preferred_element_type=jnp.float32)
            acc = d if acc is None else acc + d
        write_out(acc)

    @pl.when(m != 0)
    def _():
        t = t_ref[...].astype(jnp.bfloat16)
        write_out(jnp.dot(t, xt_ref[...],
                          preferred_element_type=jnp.float32))


@jax.jit
def kernel(x, transform):
    B, V_in, C = x.shape
    V_out = transform.shape[0]
    N = B * C

    return pl.pallas_call(
        _body,
        grid=(V_out // BM,),
        in_specs=[
            pl.BlockSpec((BM, V_in), lambda m: (m, 0)),
            pl.BlockSpec(memory_space=pltpu.MemorySpace.HBM),
        ],
        out_specs=pl.BlockSpec((B, BM, C), lambda m: (0, m, 0)),
        out_shape=jax.ShapeDtypeStruct((B, V_out, C), jnp.float32),
        scratch_shapes=[
            pltpu.VMEM((V_in, N), jnp.bfloat16),
            pltpu.VMEM((2, B, CK, C), jnp.float32),
            pltpu.SemaphoreType.DMA((2,)),
        ],
        compiler_params=pltpu.CompilerParams(
            dimension_semantics=("arbitrary",),
        ),
    )(transform, x)
